# baseline (speedup 1.0000x reference)
import functools

import jax
import jax.numpy as jnp
from jax import lax
from jax.experimental import pallas as pl
from jax.experimental.pallas import tpu as pltpu
from jax.experimental.pallas import tpu_sc as plsc

N = 10000
D = 128
NC = 1
NS = 16
NW = NC * NS
CHUNK = 128
GRP = 16
N_PAD = 10240
RPT = N_PAD // NS


def _deg_body(dst_hbm, z_hbm, ones_hbm, out_hbm, idx_v, ones_v, acc_sh):
    cid = lax.axis_index("c")
    sid = lax.axis_index("s")
    zb = sid * RPT
    pltpu.sync_copy(z_hbm.at[pl.ds(zb, RPT)], acc_sh.at[pl.ds(zb, RPT)])
    pltpu.sync_copy(ones_hbm, ones_v)
    plsc.subcore_barrier()

    def group(g, carry):
        pltpu.sync_copy(dst_hbm.at[cid, sid].at[g], idx_v)

        def step(j, c):
            pltpu.sync_copy(ones_v, acc_sh.at[idx_v.at[j]], add=True)
            return c

        lax.fori_loop(0, GRP, step, 0)
        return carry

    lax.fori_loop(0, dst_hbm.shape[2], group, 0)
    plsc.subcore_barrier()
    pltpu.sync_copy(acc_sh.at[pl.ds(zb, RPT)], out_hbm.at[cid, pl.ds(zb, RPT)])


def _agg_body(hp_hbm, src_hbm, dst_hbm, z_hbm, out_hbm,
              sidx_v, didx_v, rows0_v, rows1_v, acc_sh, semg0, semg1):
    cid = lax.axis_index("c")
    sid = lax.axis_index("s")
    zb = sid * RPT
    pltpu.sync_copy(z_hbm.at[pl.ds(zb, RPT)], acc_sh.at[pl.ds(zb, RPT)])
    plsc.subcore_barrier()

    n_groups = src_hbm.shape[2]
    gh = GRP // 2

    def group(g, carry):
        pltpu.sync_copy(src_hbm.at[cid, sid].at[g], sidx_v)
        pltpu.sync_copy(dst_hbm.at[cid, sid].at[g], didx_v)
        pltpu.async_copy(hp_hbm.at[sidx_v.at[0]], rows0_v, semg0)
        pltpu.async_copy(hp_hbm.at[sidx_v.at[1]], rows1_v, semg1)

        def step(k, c):
            j0 = 2 * k
            j1 = j0 + 1
            pltpu.make_async_copy(
                hp_hbm.at[sidx_v.at[j0]], rows0_v, semg0).wait()
            pltpu.sync_copy(rows0_v, acc_sh.at[didx_v.at[j0]], add=True)

            @pl.when(k + 1 < gh)
            def _():
                pltpu.async_copy(hp_hbm.at[sidx_v.at[j0 + 2]], rows0_v, semg0)

            pltpu.make_async_copy(
                hp_hbm.at[sidx_v.at[j1]], rows1_v, semg1).wait()
            pltpu.sync_copy(rows1_v, acc_sh.at[didx_v.at[j1]], add=True)

            @pl.when(k + 1 < gh)
            def _():
                pltpu.async_copy(hp_hbm.at[sidx_v.at[j1 + 2]], rows1_v, semg1)

            return c

        lax.fori_loop(0, gh, step, 0)
        return carry

    lax.fori_loop(0, n_groups, group, 0)
    plsc.subcore_barrier()
    pltpu.sync_copy(acc_sh.at[pl.ds(zb, RPT)], out_hbm.at[cid, pl.ds(zb, RPT)])


def _tc_pre(x_ref, w_ref, degp_ref, hp_ref, dis_ref):
    deg = sum(degp_ref[c, :, 0] for c in range(NC)) + 1.0
    dis = lax.rsqrt(deg)
    h = jnp.dot(x_ref[:, :], w_ref[:, :], preferred_element_type=jnp.float32)
    hp_ref[:, :] = h * dis[:, None]
    dis_ref[:, :] = dis[:, None]


def _tc_mid(agg_ref, hp_ref, dis_ref, b_ref, w_ref, out_ref):
    t = (dis_ref[:, :]
         * (sum(agg_ref[c] for c in range(NC)) + hp_ref[:, :]) + b_ref[:, :])
    t = jnp.where(t > 0, t, jnp.exp(t) - 1.0)
    out_ref[:, :] = (
        jnp.dot(t, w_ref[:, :], preferred_element_type=jnp.float32)
        * dis_ref[:, :])


def _tc_post(agg_ref, hp_ref, dis_ref, b_ref, out_ref):
    out_ref[:, :] = (
        dis_ref[:, :]
        * (sum(agg_ref[c] for c in range(NC)) + hp_ref[:, :]) + b_ref[:, :])


_R = 2000
_G = N // _R


def _tc_pre_call(x, w, degp):
    return pl.pallas_call(
        _tc_pre,
        grid=(_G,),
        in_specs=[
            pl.BlockSpec((_R, D), lambda i: (i, 0)),
            pl.BlockSpec((D, D), lambda i: (0, 0)),
            pl.BlockSpec((NC, _R, D), lambda i: (0, i, 0)),
        ],
        out_specs=[
            pl.BlockSpec((_R, D), lambda i: (i, 0)),
            pl.BlockSpec((_R, 1), lambda i: (i, 0)),
        ],
        out_shape=[
            jax.ShapeDtypeStruct((N, D), jnp.float32),
            jax.ShapeDtypeStruct((N, 1), jnp.float32),
        ],
    )(x, w, degp)


def _tc_mid_call(agg, hp, dis, b, w):
    return pl.pallas_call(
        _tc_mid,
        grid=(_G,),
        in_specs=[
            pl.BlockSpec((NC, _R, D), lambda i: (0, i, 0)),
            pl.BlockSpec((_R, D), lambda i: (i, 0)),
            pl.BlockSpec((_R, 1), lambda i: (i, 0)),
            pl.BlockSpec((1, D), lambda i: (0, 0)),
            pl.BlockSpec((D, D), lambda i: (0, 0)),
        ],
        out_specs=pl.BlockSpec((_R, D), lambda i: (i, 0)),
        out_shape=jax.ShapeDtypeStruct((N, D), jnp.float32),
    )(agg, hp, dis, b, w)


def _tc_post_call(agg, hp, dis, b):
    return pl.pallas_call(
        _tc_post,
        grid=(_G,),
        in_specs=[
            pl.BlockSpec((NC, _R, D), lambda i: (0, i, 0)),
            pl.BlockSpec((_R, D), lambda i: (i, 0)),
            pl.BlockSpec((_R, 1), lambda i: (i, 0)),
            pl.BlockSpec((1, D), lambda i: (0, 0)),
        ],
        out_specs=pl.BlockSpec((_R, D), lambda i: (i, 0)),
        out_shape=jax.ShapeDtypeStruct((N, D), jnp.float32),
    )(agg, hp, dis, b)


def kernel(x, edge_index, W1, b1, W2, b2, W3, b3):
    src = edge_index[0]
    dst = edge_index[1]
    e = src.shape[0]
    n_chunks = -(-e // (NW * CHUNK * GRP)) * GRP
    pad = NW * CHUNK * n_chunks - e
    n_groups = n_chunks // GRP
    src_r = jnp.concatenate(
        [src, jnp.zeros((pad,), jnp.int32)]).reshape(
            NC, NS, n_groups, GRP, CHUNK)
    pad_dst = N + (jnp.arange(pad, dtype=jnp.int32) % (N_PAD - N))
    dst_r = jnp.concatenate([dst, pad_dst]).reshape(
        NC, NS, n_groups, GRP, CHUNK)

    z128 = jnp.zeros((N_PAD, D), jnp.float32)
    o128 = jnp.ones((CHUNK, D), jnp.float32)

    mesh = plsc.VectorSubcoreMesh(core_axis_name="c", subcore_axis_name="s",
                                  num_cores=NC, num_subcores=NS)

    deg_call = pl.kernel(
        _deg_body,
        out_type=jax.ShapeDtypeStruct((NC, N_PAD, D), jnp.float32),
        mesh=mesh,
        scratch_types=[
            pltpu.VMEM((GRP, CHUNK), jnp.int32),
            pltpu.VMEM((CHUNK, D), jnp.float32),
            pltpu.VMEM_SHARED((N_PAD, D), jnp.float32),
        ],
    )

    agg_call = pl.kernel(
        _agg_body,
        out_type=jax.ShapeDtypeStruct((NC, N_PAD, D), jnp.float32),
        mesh=mesh,
        scratch_types=[
            pltpu.VMEM((GRP, CHUNK), jnp.int32),
            pltpu.VMEM((GRP, CHUNK), jnp.int32),
            pltpu.VMEM((CHUNK, D), jnp.float32),
            pltpu.VMEM((CHUNK, D), jnp.float32),
            pltpu.VMEM_SHARED((N_PAD, D), jnp.float32),
            pltpu.SemaphoreType.DMA,
            pltpu.SemaphoreType.DMA,
        ],
    )

    degp = deg_call(dst_r, z128, o128)
    hp1, dis = _tc_pre_call(x, W1, degp)
    agg1 = agg_call(hp1, src_r, dst_r, z128)
    hp2 = _tc_mid_call(agg1, hp1, dis, b1.reshape(1, D), W2)
    agg2 = agg_call(hp2, src_r, dst_r, z128)
    hp3 = _tc_mid_call(agg2, hp2, dis, b2.reshape(1, D), W3)
    agg3 = agg_call(hp3, src_r, dst_r, z128)
    return _tc_post_call(agg3, hp3, dis, b3.reshape(1, D))


# speedup vs baseline: 1.6977x; 1.6977x over previous
import functools

import jax
import jax.numpy as jnp
from jax import lax
from jax.experimental import pallas as pl
from jax.experimental.pallas import tpu as pltpu
from jax.experimental.pallas import tpu_sc as plsc

N = 10000
D = 128
NC = 2
NS = 16
NW = NC * NS
CHUNK = 128
N_PAD = 10240
RPT = N_PAD // NS


def _deg_body(dst_hbm, z_hbm, ones_hbm, out_hbm, idx_v, ones_v, acc_sh):
    cid = lax.axis_index("c")
    sid = lax.axis_index("s")
    zb = sid * RPT
    pltpu.sync_copy(z_hbm.at[pl.ds(zb, RPT)], acc_sh.at[pl.ds(zb, RPT)])
    pltpu.sync_copy(ones_hbm, ones_v)
    pltpu.sync_copy(dst_hbm.at[cid, sid], idx_v)
    plsc.subcore_barrier()

    def step(j, carry):
        pltpu.sync_copy(ones_v, acc_sh.at[idx_v.at[j]], add=True)
        return carry

    lax.fori_loop(0, idx_v.shape[0], step, 0)
    plsc.subcore_barrier()
    pltpu.sync_copy(acc_sh.at[pl.ds(zb, RPT)], out_hbm.at[cid, pl.ds(zb, RPT)])


def _agg_body(hp_hbm, src_hbm, dst_hbm, z_hbm, out_hbm,
              sidx_v, didx_v, rows_v, acc_sh, sem):
    cid = lax.axis_index("c")
    sid = lax.axis_index("s")
    zb = sid * RPT
    pltpu.sync_copy(z_hbm.at[pl.ds(zb, RPT)], acc_sh.at[pl.ds(zb, RPT)])
    pltpu.sync_copy(src_hbm.at[cid, sid], sidx_v)
    pltpu.sync_copy(dst_hbm.at[cid, sid], didx_v)
    plsc.subcore_barrier()

    def step(j, carry):
        pltpu.async_copy(hp_hbm.at[sidx_v.at[j]], rows_v, sem).wait()
        pltpu.sync_copy(rows_v, acc_sh.at[didx_v.at[j]], add=True)
        return carry

    lax.fori_loop(0, sidx_v.shape[0], step, 0)
    plsc.subcore_barrier()
    pltpu.sync_copy(acc_sh.at[pl.ds(zb, RPT)], out_hbm.at[cid, pl.ds(zb, RPT)])


def _tc_pre(x_ref, w_ref, degp_ref, hp_ref, dis_ref):
    deg = degp_ref[0, :, 0] + degp_ref[1, :, 0] + 1.0
    dis = lax.rsqrt(deg)
    h = jnp.dot(x_ref[:, :], w_ref[:, :], preferred_element_type=jnp.float32)
    hp_ref[:, :] = h * dis[:, None]
    dis_ref[:, :] = dis[:, None]


def _tc_mid(agg_ref, hp_ref, dis_ref, b_ref, w_ref, out_ref):
    t = dis_ref[:, :] * (agg_ref[0] + agg_ref[1] + hp_ref[:, :]) + b_ref[:, :]
    t = jnp.where(t > 0, t, jnp.exp(t) - 1.0)
    out_ref[:, :] = (
        jnp.dot(t, w_ref[:, :], preferred_element_type=jnp.float32)
        * dis_ref[:, :])


def _tc_post(agg_ref, hp_ref, dis_ref, b_ref, out_ref):
    out_ref[:, :] = (
        dis_ref[:, :] * (agg_ref[0] + agg_ref[1] + hp_ref[:, :]) + b_ref[:, :])


_R = 2000
_G = N // _R


def _tc_pre_call(x, w, degp):
    return pl.pallas_call(
        _tc_pre,
        grid=(_G,),
        in_specs=[
            pl.BlockSpec((_R, D), lambda i: (i, 0)),
            pl.BlockSpec((D, D), lambda i: (0, 0)),
            pl.BlockSpec((NC, _R, 16), lambda i: (0, i, 0)),
        ],
        out_specs=[
            pl.BlockSpec((_R, D), lambda i: (i, 0)),
            pl.BlockSpec((_R, 1), lambda i: (i, 0)),
        ],
        out_shape=[
            jax.ShapeDtypeStruct((N, D), jnp.float32),
            jax.ShapeDtypeStruct((N, 1), jnp.float32),
        ],
    )(x, w, degp)


def _tc_mid_call(agg, hp, dis, b, w):
    return pl.pallas_call(
        _tc_mid,
        grid=(_G,),
        in_specs=[
            pl.BlockSpec((NC, _R, D), lambda i: (0, i, 0)),
            pl.BlockSpec((_R, D), lambda i: (i, 0)),
            pl.BlockSpec((_R, 1), lambda i: (i, 0)),
            pl.BlockSpec((1, D), lambda i: (0, 0)),
            pl.BlockSpec((D, D), lambda i: (0, 0)),
        ],
        out_specs=pl.BlockSpec((_R, D), lambda i: (i, 0)),
        out_shape=jax.ShapeDtypeStruct((N, D), jnp.float32),
    )(agg, hp, dis, b, w)


def _tc_post_call(agg, hp, dis, b):
    return pl.pallas_call(
        _tc_post,
        grid=(_G,),
        in_specs=[
            pl.BlockSpec((NC, _R, D), lambda i: (0, i, 0)),
            pl.BlockSpec((_R, D), lambda i: (i, 0)),
            pl.BlockSpec((_R, 1), lambda i: (i, 0)),
            pl.BlockSpec((1, D), lambda i: (0, 0)),
        ],
        out_specs=pl.BlockSpec((_R, D), lambda i: (i, 0)),
        out_shape=jax.ShapeDtypeStruct((N, D), jnp.float32),
    )(agg, hp, dis, b)


def kernel(x, edge_index, W1, b1, W2, b2, W3, b3):
    src = edge_index[0]
    dst = edge_index[1]
    e = src.shape[0]
    n_chunks = -(-e // (NW * CHUNK))
    pad = NW * CHUNK * n_chunks - e
    src_r = jnp.concatenate(
        [src, jnp.zeros((pad,), jnp.int32)]).reshape(NC, NS, n_chunks, CHUNK)
    pad_dst = N + (jnp.arange(pad, dtype=jnp.int32) % (N_PAD - N))
    dst_r = jnp.concatenate([dst, pad_dst]).reshape(NC, NS, n_chunks, CHUNK)

    z128 = jnp.zeros((N_PAD, D), jnp.float32)
    z16 = jnp.zeros((N_PAD, 16), jnp.float32)
    o16 = jnp.ones((CHUNK, 16), jnp.float32)

    mesh = plsc.VectorSubcoreMesh(core_axis_name="c", subcore_axis_name="s",
                                  num_cores=NC, num_subcores=NS)

    deg_call = pl.kernel(
        _deg_body,
        out_type=jax.ShapeDtypeStruct((NC, N_PAD, 16), jnp.float32),
        mesh=mesh,
        scratch_types=[
            pltpu.VMEM((n_chunks, CHUNK), jnp.int32),
            pltpu.VMEM((CHUNK, 16), jnp.float32),
            pltpu.VMEM_SHARED((N_PAD, 16), jnp.float32),
        ],
    )

    agg_call = pl.kernel(
        _agg_body,
        out_type=jax.ShapeDtypeStruct((NC, N_PAD, D), jnp.float32),
        mesh=mesh,
        scratch_types=[
            pltpu.VMEM((n_chunks, CHUNK), jnp.int32),
            pltpu.VMEM((n_chunks, CHUNK), jnp.int32),
            pltpu.VMEM((CHUNK, D), jnp.float32),
            pltpu.VMEM_SHARED((N_PAD, D), jnp.float32),
            pltpu.SemaphoreType.DMA,
        ],
    )

    degp = deg_call(dst_r, z16, o16)
    hp1, dis = _tc_pre_call(x, W1, degp)
    agg1 = agg_call(hp1, src_r, dst_r, z128)
    hp2 = _tc_mid_call(agg1, hp1, dis, b1.reshape(1, D), W2)
    agg2 = agg_call(hp2, src_r, dst_r, z128)
    hp3 = _tc_mid_call(agg2, hp2, dis, b2.reshape(1, D), W3)
    agg3 = agg_call(hp3, src_r, dst_r, z128)
    return _tc_post_call(agg3, hp3, dis, b3.reshape(1, D))
